# baseline (speedup 1.0000x reference)
import jax
import jax.numpy as jnp
from jax.experimental import pallas as pl
from jax.experimental.pallas import tpu as pltpu

F_OUT = 32


def _gclstm_block(x_ref, h_ref, c_ref, wi_ref, wf_ref, wc_ref, wo_ref,
                  ci_ref, cf_ref, cc_ref, co_ref, cib_ref, cfb_ref, ccb_ref,
                  cob_ref, wci_ref, wcf_ref, wco_ref, bi_ref, bf_ref, bc_ref,
                  bo_ref, linw_ref, linb_ref, out_ref, h_out_ref, c_out_ref):
    f4 = 4 * F_OUT
    wp = jnp.concatenate(
        [wi_ref[...], wf_ref[...], wc_ref[...], wo_ref[...]], axis=1)
    cp = jnp.concatenate(
        [ci_ref[...], cf_ref[...], cc_ref[...], co_ref[...]], axis=1)
    x4 = x_ref[...]
    h4 = h_ref[...]
    c4 = c_ref[...]
    gj = [jnp.dot(x4[:, 512 * j // 4:512 * j // 4 + 128], wp,
                  preferred_element_type=jnp.float32)
          + jnp.dot(h4[:, F_OUT * j:F_OUT * (j + 1)], cp,
                    preferred_element_type=jnp.float32)
          for j in range(4)]
    g4 = jnp.concatenate(
        [gj[j][:, F_OUT * g:F_OUT * (g + 1)] for g in range(4)
         for j in range(4)], axis=1)
    bias4 = jnp.concatenate(
        [jnp.concatenate([blk] * 4, axis=1) for blk in
         (cib_ref[...] + bi_ref[...], cfb_ref[...] + bf_ref[...],
          ccb_ref[...] + bc_ref[...], cob_ref[...] + bo_ref[...])], axis=1)
    g4 = g4 + bias4
    wci4 = jnp.concatenate([wci_ref[...]] * 4, axis=1)
    wcf4 = jnp.concatenate([wcf_ref[...]] * 4, axis=1)
    wco4 = jnp.concatenate([wco_ref[...]] * 4, axis=1)
    i_g = jax.nn.sigmoid(g4[:, 0 * f4:1 * f4] + wci4 * c4)
    f_g = jax.nn.sigmoid(g4[:, 1 * f4:2 * f4] + wcf4 * c4)
    t_g = jnp.tanh(g4[:, 2 * f4:3 * f4])
    c_new = f_g * c4 + i_g * t_g
    o_g = jax.nn.sigmoid(g4[:, 3 * f4:4 * f4] + wco4 * c_new)
    h_new = o_g * jnp.tanh(c_new)
    c_out_ref[...] = c_new
    h_out_ref[...] = h_new
    row_id = jax.lax.broadcasted_iota(jnp.int32, (f4, 4), 0)
    col_id = jax.lax.broadcasted_iota(jnp.int32, (f4, 4), 1)
    lin_tile = jnp.concatenate([linw_ref[...]] * 4, axis=0)
    lin4 = jnp.where(row_id // F_OUT == col_id, lin_tile, 0.0)
    out_ref[...] = (jnp.dot(h_new, lin4, preferred_element_type=jnp.float32)
                    + linb_ref[...])


def kernel(x, edge_index, edge_weight, h, c, W_i, W_f, W_c, W_o, conv_i_w,
           conv_i_b, conv_f_w, conv_f_b, conv_c_w, conv_c_b, conv_o_w,
           conv_o_b, w_c_i, w_c_f, w_c_o, b_i, b_f, b_c, b_o, lin_w, lin_b):
    del edge_index, edge_weight
    n, f_in = x.shape
    f_out = h.shape[1]
    rows = n // 4

    x4 = x.reshape(rows, 4 * f_in)
    h4 = h.reshape(rows, 4 * f_out)
    c4 = c.reshape(rows, 4 * f_out)
    cib = conv_i_b.reshape(1, f_out)
    cfb = conv_f_b.reshape(1, f_out)
    ccb = conv_c_b.reshape(1, f_out)
    cob = conv_o_b.reshape(1, f_out)
    linb = lin_b.reshape(1, 1)

    full_spec = lambda a: pl.BlockSpec(a.shape, lambda: (0, 0))

    out4, h_new4, c_new4 = pl.pallas_call(
        _gclstm_block,
        in_specs=[
            full_spec(x4), full_spec(h4), full_spec(c4),
            full_spec(W_i), full_spec(W_f), full_spec(W_c), full_spec(W_o),
            full_spec(conv_i_w), full_spec(conv_f_w), full_spec(conv_c_w),
            full_spec(conv_o_w),
            full_spec(cib), full_spec(cfb), full_spec(ccb), full_spec(cob),
            full_spec(w_c_i), full_spec(w_c_f), full_spec(w_c_o),
            full_spec(b_i), full_spec(b_f), full_spec(b_c), full_spec(b_o),
            full_spec(lin_w), full_spec(linb),
        ],
        out_specs=[
            pl.BlockSpec((rows, 4), lambda: (0, 0)),
            pl.BlockSpec((rows, 4 * f_out), lambda: (0, 0)),
            pl.BlockSpec((rows, 4 * f_out), lambda: (0, 0)),
        ],
        out_shape=[
            jax.ShapeDtypeStruct((rows, 4), jnp.float32),
            jax.ShapeDtypeStruct((rows, 4 * f_out), jnp.float32),
            jax.ShapeDtypeStruct((rows, 4 * f_out), jnp.float32),
        ],
    )(x4, h4, c4, W_i, W_f, W_c, W_o, conv_i_w, conv_f_w, conv_c_w, conv_o_w,
      cib, cfb, ccb, cob, w_c_i, w_c_f, w_c_o, b_i, b_f, b_c, b_o,
      lin_w, linb)
    return (out4.reshape(n, 1), h_new4.reshape(n, f_out),
            c_new4.reshape(n, f_out))
